# baseline (speedup 1.0000x reference)
import functools

import jax
import jax.numpy as jnp
from jax import lax
from jax.experimental import pallas as pl
from jax.experimental.pallas import tpu as pltpu
from jax.experimental.pallas import tpu_sc as plsc

_NUM_CORES = 2
_NUM_SUBCORES = 16
_NUM_WORKERS = _NUM_CORES * _NUM_SUBCORES
_CHUNK = 512


@jax.jit
def _sc_gather(embeddings, x):
    L, _, B = x.shape
    n = L * B
    dim = embeddings.shape[1]
    per_worker = n // _NUM_WORKERS
    nchunks = per_worker // _CHUNK
    npairs = nchunks // 2
    obuf_rows = _CHUNK * dim // 128
    jb_rows = obuf_rows // 4
    mesh = plsc.VectorSubcoreMesh(core_axis_name="c", subcore_axis_name="s")

    @functools.partial(
        pl.kernel,
        mesh=mesh,
        out_type=jax.ShapeDtypeStruct((n * dim // 128, 128), jnp.float32),
        compiler_params=pltpu.CompilerParams(
            use_tc_tiling_on_sc=False, needs_layout_passes=False
        ),
        scratch_types=[
            pltpu.VMEM((_CHUNK,), jnp.int32),
            pltpu.VMEM((_CHUNK,), jnp.int32),
            pltpu.VMEM((_CHUNK, dim), jnp.float32),
            pltpu.VMEM((_CHUNK, dim), jnp.float32),
            pltpu.VMEM((obuf_rows, 128), jnp.float32),
            pltpu.VMEM((obuf_rows, 128), jnp.float32),
            pltpu.SemaphoreType.DMA,
            pltpu.SemaphoreType.DMA,
            pltpu.SemaphoreType.DMA,
            pltpu.SemaphoreType.DMA,
        ],
    )
    def k(table_hbm, x_hbm, out_hbm, i0, i1, r0, r1, o0, o1, g0, g1, w0, w1):
        wid = lax.axis_index("s") * _NUM_CORES + lax.axis_index("c")
        base = wid * per_worker
        bufs = ((i0, r0, o0, g0, w0), (i1, r1, o1, g1, w1))
        iota16 = lax.iota(jnp.int32, 16)

        def fire(c, b):
            idx_v, rows_v, _, gsem, _ = bufs[b]
            off = base + c * _CHUNK
            pltpu.sync_copy(x_hbm.at[off // B, 0, pl.ds(off % B, _CHUNK)], idx_v)
            pltpu.async_copy(table_hbm.at[idx_v], rows_v, gsem)

        def drain_gather(b):
            idx_v, rows_v, _, gsem, _ = bufs[b]
            pltpu.make_async_copy(table_hbm.at[idx_v], rows_v, gsem).wait()

        def permute_and_write(c, b):
            _, rows_v, obuf, _, wsem = bufs[b]
            off = base + c * _CHUNK
            l = off // B
            bt0 = (off % B) // 128

            @pl.loop(0, obuf_rows)
            def _(r):
                jb = r // jb_rows
                rem = r % jb_rows
                btl = rem // 8
                js = rem % 8
                col16 = jnp.full((16,), jb * 8 + js, jnp.int32)
                row_off = btl * 128
                for kg in range(8):
                    vals = plsc.load_gather(
                        rows_v, [row_off + kg * 16 + iota16, col16]
                    )
                    obuf[r, pl.ds(kg * 16, 16)] = vals

            for jb in range(4):
                r0_ = l * 1024 + jb * 256 + bt0 * 8
                pltpu.async_copy(
                    obuf.at[pl.ds(jb * jb_rows, jb_rows), :],
                    out_hbm.at[pl.ds(r0_, jb_rows), :],
                    wsem,
                )

        def drain_write(b):
            _, _, obuf, _, wsem = bufs[b]
            pltpu.make_async_copy(
                obuf, out_hbm.at[pl.ds(0, obuf_rows), :], wsem
            ).wait()

        fire(0, 0)

        @pl.loop(0, npairs)
        def _(g):
            c0 = 2 * g

            @pl.when(g > 0)
            def _():
                drain_write(1)

            fire(c0 + 1, 1)
            drain_gather(0)

            @pl.when(g > 0)
            def _():
                drain_write(0)

            permute_and_write(c0, 0)

            @pl.when(g < npairs - 1)
            def _():
                fire(c0 + 2, 0)

            drain_gather(1)
            permute_and_write(c0 + 1, 1)

        drain_write(0)
        drain_write(1)

    out_lin = k(embeddings, x)
    view = out_lin.reshape(L, dim // 8, B // 128, 8, 128)
    return view.transpose(0, 2, 4, 1, 3).reshape(L, B, dim)


def kernel(x, embeddings):
    return _sc_gather(embeddings, x)
